# baseline (speedup 1.0000x reference)
import functools

import jax
import jax.numpy as jnp
from jax import lax
from jax.experimental import pallas as pl
from jax.experimental.pallas import tpu as pltpu
from jax.experimental.pallas import tpu_sc as plsc

N_COARSE = 10000
N_FINE = 20000
E = 320000
C_BRANCH = 128
C_IN = 256
C_OUT = 128

NC = 2
NS = 16

C_HALF = C_OUT // NC
EPT = 20480
E_PAD = NS * EPT
IDX_ROWS = EPT // 128
K = 4
HALF = K * 128
NPAIRS = EPT // (2 * HALF)
PAIR_ROWS = 2 * K
FPT_REAL = N_FINE // NS
U_ROWS = 10
H_ROWS = N_COARSE + 16
INIT_ROWS = 640

BN = 2000
NB = N_COARSE // BN


def _dense_body(at_ref, bt_ref, wn_ref, ws_ref, bias_ref, y_ref, s_ref):
    xt = jnp.concatenate([at_ref[...], bt_ref[...]], axis=1)
    y_ref[0] = jnp.dot(xt, wn_ref[0], preferred_element_type=jnp.float32)
    y_ref[1] = jnp.dot(xt, wn_ref[1], preferred_element_type=jnp.float32)
    s_ref[...] = (
        jnp.dot(xt, ws_ref[...], preferred_element_type=jnp.float32)
        + bias_ref[...]
    )


def _dense(at, bt, wnt3, wst, bias2):
    return pl.pallas_call(
        _dense_body,
        grid=(NB,),
        in_specs=[
            pl.BlockSpec((BN, C_BRANCH), lambda n: (n, 0)),
            pl.BlockSpec((BN, C_BRANCH), lambda n: (n, 0)),
            pl.BlockSpec((NC, C_IN, C_HALF), lambda n: (0, 0, 0)),
            pl.BlockSpec((C_IN, C_OUT), lambda n: (0, 0)),
            pl.BlockSpec((1, C_OUT), lambda n: (0, 0)),
        ],
        out_specs=[
            pl.BlockSpec((NC, BN, C_HALF), lambda n: (0, n, 0)),
            pl.BlockSpec((BN, C_OUT), lambda n: (n, 0)),
        ],
        out_shape=[
            jax.ShapeDtypeStruct((NC, N_COARSE, C_HALF), jnp.float32),
            jax.ShapeDtypeStruct((N_COARSE, C_OUT), jnp.float32),
        ],
    )(at, bt, wnt3, wst, bias2)


def _transpose_body(in_ref, out_ref):
    out_ref[...] = in_ref[...].T


def _transpose(x):
    bt = 2560
    return pl.pallas_call(
        _transpose_body,
        grid=((N_FINE + bt - 1) // bt,),
        in_specs=[pl.BlockSpec((bt, C_OUT), lambda n: (n, 0))],
        out_specs=pl.BlockSpec((C_OUT, bt), lambda n: (0, n)),
        out_shape=jax.ShapeDtypeStruct((C_OUT, N_FINE), jnp.float32),
    )(x)


def _sc_body(y_hbm, s_hbm, g2d, r2d, u3, out_hbm,
             rows, idx_g, idx_r, idx_u, h_sh, sem_g, sem_s, sem_i):
    c = lax.axis_index("c")
    t = lax.axis_index("s")
    col = c * C_HALF
    gbase = t * IDX_ROWS
    rbase = t * IDX_ROWS

    base = jnp.minimum(t * INIT_ROWS, N_COARSE - INIT_ROWS)
    pltpu.sync_copy(s_hbm.at[pl.ds(base, INIT_ROWS), pl.ds(col, C_HALF)],
                    rows.at[pl.ds(0, INIT_ROWS)])
    pltpu.sync_copy(rows.at[pl.ds(0, INIT_ROWS)],
                    h_sh.at[pl.ds(base, INIT_ROWS)])
    plsc.subcore_barrier()

    def bank(i):
        return lax.rem(i, 2) * PAIR_ROWS

    def i_cp(i, src, srcbase, dst):
        return pltpu.make_async_copy(
            src.at[pl.ds(srcbase + i * PAIR_ROWS, PAIR_ROWS)],
            dst.at[pl.ds(bank(i), PAIR_ROWS)], sem_i)

    def g_cp(i, h, j):
        return pltpu.make_async_copy(
            y_hbm.at[c].at[idx_g.at[bank(i) + h * K + j]],
            rows.at[pl.ds((h * K + j) * 128, 128)], sem_g)

    def s_cp(i, h, j):
        return pltpu.make_async_copy(
            rows.at[pl.ds((h * K + j) * 128, 128)],
            h_sh.at[idx_r.at[bank(i) + h * K + j]], sem_s)

    i_cp(0, g2d, gbase, idx_g).start()
    i_cp(0, r2d, rbase, idx_r).start()
    i_cp(0, g2d, gbase, idx_g).wait()
    i_cp(0, r2d, rbase, idx_r).wait()
    for j in range(K):
        g_cp(0, 0, j).start()

    def pair_body(i, carry):
        for j in range(K):
            g_cp(i, 0, j).wait()

        @pl.when(i > 0)
        def _():
            for j in range(K):
                s_cp(i, 1, j).wait()

        @pl.when(i < NPAIRS - 1)
        def _():
            i_cp(i + 1, g2d, gbase, idx_g).start()
            i_cp(i + 1, r2d, rbase, idx_r).start()

        for j in range(K):
            g_cp(i, 1, j).start()
        for j in range(K):
            s_cp(i, 0, j).start(add=True)
        for j in range(K):
            g_cp(i, 1, j).wait()
        for j in range(K):
            s_cp(i, 0, j).wait()

        @pl.when(i < NPAIRS - 1)
        def _():
            i_cp(i + 1, g2d, gbase, idx_g).wait()
            i_cp(i + 1, r2d, rbase, idx_r).wait()
            for j in range(K):
                g_cp(i + 1, 0, j).start()

        for j in range(K):
            s_cp(i, 1, j).start(add=True)
        return carry

    lax.fori_loop(0, NPAIRS, pair_body, 0)
    for j in range(K):
        s_cp(NPAIRS - 1, 1, j).wait()
    plsc.subcore_barrier()

    pltpu.sync_copy(u3.at[t], idx_u)
    chunks = [(0, 0, 4, 512), (4, 512, 4, 512), (8, 0, 2, FPT_REAL - 1024)]
    st_cps = []
    for ci, (urow, roff, nstream, nrows) in enumerate(chunks):
        if ci == 2:
            st_cps[0].wait()
        cps = [
            pltpu.async_copy(h_sh.at[idx_u.at[urow + j]],
                             rows.at[pl.ds(roff + j * 128, 128)], sem_g)
            for j in range(nstream)
        ]
        for cp in cps:
            cp.wait()
        st = pltpu.make_async_copy(
            rows.at[pl.ds(roff, nrows)],
            out_hbm.at[pl.ds(t * FPT_REAL + (0 if ci == 0 else (512 if ci == 1 else 1024)), nrows),
                       pl.ds(col, C_HALF)], sem_s)
        st.start()
        st_cps.append(st)
    st_cps[1].wait()
    st_cps[2].wait()


_sc_sparse = functools.partial(
    pl.kernel,
    out_type=jax.ShapeDtypeStruct((N_FINE, C_OUT), jnp.float32),
    mesh=plsc.VectorSubcoreMesh(core_axis_name="c", subcore_axis_name="s"),
    scratch_types=[
        pltpu.VMEM((2 * HALF, C_HALF), jnp.float32),
        pltpu.VMEM((2 * PAIR_ROWS, 128), jnp.int32),
        pltpu.VMEM((2 * PAIR_ROWS, 128), jnp.int32),
        pltpu.VMEM((U_ROWS, 128), jnp.int32),
        pltpu.VMEM_SHARED((H_ROWS, C_HALF), jnp.float32),
        pltpu.SemaphoreType.DMA,
        pltpu.SemaphoreType.DMA,
        pltpu.SemaphoreType.DMA,
    ],
    compiler_params=pltpu.CompilerParams(use_tc_tiling_on_sc=False),
)(_sc_body)


def kernel(a, b, W_self, W_neigh, bias, gather_index, reduce_index,
           unpool_index):
    at = a.T
    bt = b.T
    wnt3 = W_neigh.T.reshape(C_IN, NC, C_HALF).transpose(1, 0, 2)
    wst = W_self.T
    bias2 = bias.reshape(1, C_OUT)

    y3, s_full = _dense(at, bt, wnt3, wst, bias2)

    g2 = jnp.pad(gather_index, (0, E_PAD - E)).reshape(-1, 128)
    r2 = jnp.pad(reduce_index, (0, E_PAD - E),
                 constant_values=N_COARSE).reshape(-1, 128)
    u3 = jnp.pad(unpool_index.reshape(NS, FPT_REAL),
                 ((0, 0), (0, U_ROWS * 128 - FPT_REAL))).reshape(NS, U_ROWS, 128)

    out = _sc_sparse(y3, s_full, g2, r2, u3)
    return _transpose(out)


# speedup vs baseline: 1.2518x; 1.2518x over previous
import functools

import jax
import jax.numpy as jnp
from jax import lax
from jax.experimental import pallas as pl
from jax.experimental.pallas import tpu as pltpu
from jax.experimental.pallas import tpu_sc as plsc

N_COARSE = 10000
N_FINE = 20000
E = 320000
C_BRANCH = 128
C_IN = 256
C_OUT = 128

NC = 2
NS = 16

C_HALF = C_OUT // NC
EPT = 20480
E_PAD = NS * EPT
IDX_ROWS = EPT // 128
K = 4
HALF = K * 128
NPAIRS = EPT // (2 * HALF)
PAIR_ROWS = 2 * K
FPT_REAL = N_FINE // NS
U_ROWS = 10
H_ROWS = N_COARSE + 16
INIT_ROWS = 640

BN = 2000
NB = N_COARSE // BN


def _dense_body(at_ref, bt_ref, wn_ref, ws_ref, bias_ref, y_ref, s_ref):
    xt = jnp.concatenate([at_ref[...], bt_ref[...]], axis=1)
    y_ref[0] = jnp.dot(
        xt, wn_ref[0], preferred_element_type=jnp.float32).astype(jnp.bfloat16)
    y_ref[1] = jnp.dot(
        xt, wn_ref[1], preferred_element_type=jnp.float32).astype(jnp.bfloat16)
    s_ref[...] = (
        jnp.dot(xt, ws_ref[...], preferred_element_type=jnp.float32)
        + bias_ref[...]
    ).astype(jnp.bfloat16)


def _dense(at, bt, wnt3, wst, bias2):
    return pl.pallas_call(
        _dense_body,
        grid=(NB,),
        in_specs=[
            pl.BlockSpec((BN, C_BRANCH), lambda n: (n, 0)),
            pl.BlockSpec((BN, C_BRANCH), lambda n: (n, 0)),
            pl.BlockSpec((NC, C_IN, C_HALF), lambda n: (0, 0, 0)),
            pl.BlockSpec((C_IN, C_OUT), lambda n: (0, 0)),
            pl.BlockSpec((1, C_OUT), lambda n: (0, 0)),
        ],
        out_specs=[
            pl.BlockSpec((NC, BN, C_HALF), lambda n: (0, n, 0)),
            pl.BlockSpec((BN, C_OUT), lambda n: (n, 0)),
        ],
        out_shape=[
            jax.ShapeDtypeStruct((NC, N_COARSE, C_HALF), jnp.bfloat16),
            jax.ShapeDtypeStruct((N_COARSE, C_OUT), jnp.bfloat16),
        ],
    )(at, bt, wnt3, wst, bias2)


def _transpose_body(in_ref, out_ref):
    out_ref[...] = in_ref[...].astype(jnp.float32).T


def _transpose(x):
    bt = 2560
    return pl.pallas_call(
        _transpose_body,
        grid=((N_FINE + bt - 1) // bt,),
        in_specs=[pl.BlockSpec((bt, C_OUT), lambda n: (n, 0))],
        out_specs=pl.BlockSpec((C_OUT, bt), lambda n: (0, n)),
        out_shape=jax.ShapeDtypeStruct((C_OUT, N_FINE), jnp.float32),
    )(x)


def _sc_body(y_hbm, s_hbm, g2d, r2d, u3, out_hbm,
             rows, idx_g, idx_r, idx_u, h_sh, sem_g, sem_s, sem_i):
    c = lax.axis_index("c")
    t = lax.axis_index("s")
    col = c * C_HALF
    gbase = (c * NS + t) * IDX_ROWS
    rbase = t * IDX_ROWS

    base = jnp.minimum(t * INIT_ROWS, N_COARSE - INIT_ROWS)
    pltpu.sync_copy(s_hbm.at[pl.ds(base, INIT_ROWS), pl.ds(col, C_HALF)],
                    rows.at[pl.ds(0, INIT_ROWS)])
    pltpu.sync_copy(rows.at[pl.ds(0, INIT_ROWS)],
                    h_sh.at[pl.ds(base, INIT_ROWS)])
    plsc.subcore_barrier()

    def bank(i):
        return lax.rem(i, 2) * PAIR_ROWS

    def i_cp(i, src, srcbase, dst):
        return pltpu.make_async_copy(
            src.at[pl.ds(srcbase + i * PAIR_ROWS, PAIR_ROWS)],
            dst.at[pl.ds(bank(i), PAIR_ROWS)], sem_i)

    def g_cp(i, h, j):
        return pltpu.make_async_copy(
            y_hbm.at[idx_g.at[bank(i) + h * K + j]],
            rows.at[pl.ds((h * K + j) * 128, 128)], sem_g)

    def s_cp(i, h, j):
        return pltpu.make_async_copy(
            rows.at[pl.ds((h * K + j) * 128, 128)],
            h_sh.at[idx_r.at[bank(i) + h * K + j]], sem_s)

    i_cp(0, g2d, gbase, idx_g).start()
    i_cp(0, r2d, rbase, idx_r).start()
    i_cp(0, g2d, gbase, idx_g).wait()
    i_cp(0, r2d, rbase, idx_r).wait()
    for j in range(K):
        g_cp(0, 0, j).start()

    def pair_body(i, carry):
        for j in range(K):
            g_cp(i, 0, j).wait()

        @pl.when(i > 0)
        def _():
            for j in range(K):
                s_cp(i, 1, j).wait()

        @pl.when(i < NPAIRS - 1)
        def _():
            i_cp(i + 1, g2d, gbase, idx_g).start()
            i_cp(i + 1, r2d, rbase, idx_r).start()

        for j in range(K):
            g_cp(i, 1, j).start()
        for j in range(K):
            s_cp(i, 0, j).start(add=True)
        for j in range(K):
            g_cp(i, 1, j).wait()
        for j in range(K):
            s_cp(i, 0, j).wait()

        @pl.when(i < NPAIRS - 1)
        def _():
            i_cp(i + 1, g2d, gbase, idx_g).wait()
            i_cp(i + 1, r2d, rbase, idx_r).wait()
            for j in range(K):
                g_cp(i + 1, 0, j).start()

        for j in range(K):
            s_cp(i, 1, j).start(add=True)
        return carry

    lax.fori_loop(0, NPAIRS, pair_body, 0)
    for j in range(K):
        s_cp(NPAIRS - 1, 1, j).wait()
    plsc.subcore_barrier()

    pltpu.sync_copy(u3.at[t], idx_u)
    chunks = [(0, 0, 4, 512), (4, 512, 4, 512), (8, 0, 2, FPT_REAL - 1024)]
    st_cps = []
    for ci, (urow, roff, nstream, nrows) in enumerate(chunks):
        if ci == 2:
            st_cps[0].wait()
        cps = [
            pltpu.async_copy(h_sh.at[idx_u.at[urow + j]],
                             rows.at[pl.ds(roff + j * 128, 128)], sem_g)
            for j in range(nstream)
        ]
        for cp in cps:
            cp.wait()
        st = pltpu.make_async_copy(
            rows.at[pl.ds(roff, nrows)],
            out_hbm.at[pl.ds(t * FPT_REAL + (0 if ci == 0 else (512 if ci == 1 else 1024)), nrows),
                       pl.ds(col, C_HALF)], sem_s)
        st.start()
        st_cps.append(st)
    st_cps[1].wait()
    st_cps[2].wait()


_sc_sparse = functools.partial(
    pl.kernel,
    out_type=jax.ShapeDtypeStruct((N_FINE, C_OUT), jnp.bfloat16),
    mesh=plsc.VectorSubcoreMesh(core_axis_name="c", subcore_axis_name="s"),
    scratch_types=[
        pltpu.VMEM((2 * HALF, C_HALF), jnp.bfloat16),
        pltpu.VMEM((2 * PAIR_ROWS, 128), jnp.int32),
        pltpu.VMEM((2 * PAIR_ROWS, 128), jnp.int32),
        pltpu.VMEM((U_ROWS, 128), jnp.int32),
        pltpu.VMEM_SHARED((H_ROWS, C_HALF), jnp.bfloat16),
        pltpu.SemaphoreType.DMA,
        pltpu.SemaphoreType.DMA,
        pltpu.SemaphoreType.DMA,
    ],
    compiler_params=pltpu.CompilerParams(use_tc_tiling_on_sc=False),
)(_sc_body)


def kernel(a, b, W_self, W_neigh, bias, gather_index, reduce_index,
           unpool_index):
    at = a.T
    bt = b.T
    wnt3 = W_neigh.T.reshape(C_IN, NC, C_HALF).transpose(1, 0, 2)
    wst = W_self.T
    bias2 = bias.reshape(1, C_OUT)

    y3, s_full = _dense(at, bt, wnt3, wst, bias2)
    y_stack = y3.reshape(NC * N_COARSE, C_HALF)

    g_pad = jnp.pad(gather_index, (0, E_PAD - E))
    g2 = jnp.concatenate([g_pad, g_pad + N_COARSE]).reshape(-1, 128)
    r2 = jnp.pad(reduce_index, (0, E_PAD - E),
                 constant_values=N_COARSE).reshape(-1, 128)
    u3 = jnp.pad(unpool_index.reshape(NS, FPT_REAL),
                 ((0, 0), (0, U_ROWS * 128 - FPT_REAL))).reshape(NS, U_ROWS, 128)

    out = _sc_sparse(y_stack, s_full, g2, r2, u3)
    return _transpose(out)
